# baseline (speedup 1.0000x reference)
import functools

import jax
import jax.numpy as jnp
from jax.experimental import pallas as pl
from jax.experimental.pallas import tpu as pltpu

HIDDEN_DIM = 64
MEMORY_SLOTS = 8
VOCAB_SIZE = 64
N_HEADS = 2
HEAD_DIM = HIDDEN_DIM // N_HEADS
B = 128
L = 512
BB = 16
V = VOCAB_SIZE

_TRANS_RHS = (((1,), (1,)), ((), ()))


def _dot(a, b):
    return jax.lax.dot_general(a, b, (((1,), (0,)), ((), ())),
                               preferred_element_type=jnp.float32)


def _dot_tb(a, b):
    return jax.lax.dot_general(a, b, _TRANS_RHS,
                               preferred_element_type=jnp.float32)


def _layer_norm(x, g, b):
    m = jnp.mean(x, axis=-1, keepdims=True)
    v = jnp.mean((x - m) ** 2, axis=-1, keepdims=True)
    return (x - m) * jax.lax.rsqrt(v + 1e-5) * g + b


def _expand(mat):
    return jnp.reshape(
        jnp.broadcast_to(jnp.reshape(mat, (BB, 1, V)), (BB, V, V)),
        (BB * V, V))


def _step(seq_ref, query_ref, target_ref, seg_ref, seg2_ref, eye_t_ref,
          embed_ref,
          wq0_ref, wq1_ref, wk0_ref, wk1_ref, wv0_ref, wv1_ref,
          bq0_ref, bq1_ref, bk0_ref, bk1_ref, bv0_ref, bv1_ref,
          wo0_ref, wo1_ref, bo_ref, w1_ref, b1_ref, w2_ref, b2_ref,
          n1g_ref, n1b_ref, n2g_ref, n2b_ref, gate_ref,
          qemb_ref, wqp_ref, bqp_ref, wop_ref, bop_ref, out_ref):
    i = pl.program_id(0)

    @pl.when(i == 0)
    def _():
        out_ref[...] = jnp.zeros_like(out_ref)

    BL = BB * L
    BV = BB * V
    f32 = jnp.float32
    iota_v = jax.lax.broadcasted_iota(jnp.int32, (BL, V), 1)
    iota_bv = jax.lax.broadcasted_iota(jnp.int32, (BB, V), 1)
    ones_col = jnp.ones((V, 1), f32)
    eye_t = eye_t_ref[...]
    seg2 = seg2_ref[...]

    def rowsum(x):
        return _dot(x, ones_col)

    onehot = (seq_ref[...] == iota_v).astype(jnp.bfloat16)
    c_mat = jax.lax.dot_general(seg_ref[...], onehot, (((1,), (0,)), ((), ())),
                                preferred_element_type=f32)
    c_expand = _expand(c_mat)
    c_col = rowsum(c_expand * eye_t)

    emb = embed_ref[...]
    scale = 1.0 / (HEAD_DIM ** 0.5)

    def head_tab(wq, wk, bq, bk):
        q = _dot(emb, wq[...]) + bq[...]
        k = _dot(emb, wk[...]) + bk[...]
        return jnp.exp(_dot_tb(q * scale, k))

    expS0 = head_tab(wq0_ref, wk0_ref, bq0_ref, bk0_ref)
    expS1 = head_tab(wq1_ref, wk1_ref, bq1_ref, bk1_ref)
    v0 = _dot(emb, wv0_ref[...]) + bv0_ref[...]
    v1 = _dot(emb, wv1_ref[...]) + bv1_ref[...]

    def att(expS, vh):
        p = jnp.concatenate([expS] * BB, axis=0) * c_expand
        return _dot(p, vh) * (1.0 / rowsum(p))

    a_out = (_dot(att(expS0, v0), wo0_ref[...])
             + _dot(att(expS1, v1), wo1_ref[...]) + bo_ref[...])

    h0 = jnp.concatenate([emb] * BB, axis=0)
    h1 = _layer_norm(h0 + a_out, n1g_ref[...], n1b_ref[...])
    ff = _dot(jnp.maximum(_dot(h1, w1_ref[...]) + b1_ref[...], 0.0),
              w2_ref[...]) + b2_ref[...]
    h2 = _layer_norm(h1 + ff, n2g_ref[...], n2b_ref[...])

    s_col = rowsum(h2 * gate_ref[...])
    s_mat = _dot(seg2, s_col * eye_t)
    neg = jnp.float32(-3e38)
    sm_col = jnp.where(c_col > 0.0, s_col, neg)
    sm_exp = _expand(jnp.where(c_mat > 0.0, s_mat, neg))
    gt = (sm_exp > sm_col).astype(f32)
    before = rowsum(gt * c_expand)
    m_col = jnp.clip(jnp.float32(MEMORY_SLOTS) - before, 0.0, c_col)
    m_col = jnp.where(c_col > 0.0, m_col, 0.0)

    q_oh = (iota_bv == query_ref[...]).astype(f32)
    q_h = _dot(q_oh, qemb_ref[...])
    qp = _dot(q_h, wqp_ref[...]) + bqp_ref[...]
    s2_col = rowsum(h2 * _expand(qp)) * (1.0 / (HIDDEN_DIM ** 0.5))
    e2 = jnp.exp(s2_col) * m_col
    denom = _dot(seg2, e2)
    read = _dot(seg2, e2 * h2) * (1.0 / denom)

    logits = _dot(read, wop_ref[...]) + bop_ref[...]
    ml = jnp.max(logits, axis=1, keepdims=True)
    lse = ml + jnp.log(jnp.sum(jnp.exp(logits - ml), axis=1, keepdims=True))
    t_oh = (iota_bv == target_ref[...]).astype(f32)
    tgt = jnp.sum(logits * t_oh, axis=1, keepdims=True)
    out_ref[...] += jnp.sum(lse - tgt) * (1.0 / B)


@functools.partial(jax.jit, static_argnames=("interpret",))
def _run(seq, query, target, embed, in_proj_w, in_proj_b, attn_out_w,
         attn_out_b, ff_w1, ff_b1, ff_w2, ff_b2, norm1_g, norm1_b, norm2_g,
         norm2_b, gate_w, gate_b, q_embed, qp_w, qp_b, op_w, op_b,
         interpret=False):
    f32 = jnp.float32
    seq_col = seq.astype(jnp.int32).reshape(B * L, 1)
    query_col = query.astype(jnp.int32).reshape(B, 1)
    target_col = target.astype(jnp.int32).reshape(B, 1)
    BL, BV = BB * L, BB * V
    seg = (jnp.arange(BL, dtype=jnp.int32)[None, :] // L
           == jnp.arange(BB, dtype=jnp.int32)[:, None]).astype(jnp.bfloat16)
    seg2 = (jnp.arange(BV, dtype=jnp.int32)[None, :] // V
            == jnp.arange(BB, dtype=jnp.int32)[:, None]).astype(f32)
    eye_t = jnp.tile(jnp.eye(V, dtype=f32), (BB, 1))
    HD = HEAD_DIM
    wq0 = in_proj_w[0:HD].T
    wq1 = in_proj_w[HD:2 * HD].T
    wk0 = in_proj_w[2 * HD:3 * HD].T
    wk1 = in_proj_w[3 * HD:4 * HD].T
    wv0 = in_proj_w[4 * HD:5 * HD].T
    wv1 = in_proj_w[5 * HD:6 * HD].T
    bq0 = in_proj_b[0:HD].reshape(1, HD)
    bq1 = in_proj_b[HD:2 * HD].reshape(1, HD)
    bk0 = in_proj_b[2 * HD:3 * HD].reshape(1, HD)
    bk1 = in_proj_b[3 * HD:4 * HD].reshape(1, HD)
    bv0 = in_proj_b[4 * HD:5 * HD].reshape(1, HD)
    bv1 = in_proj_b[5 * HD:6 * HD].reshape(1, HD)
    wo0 = attn_out_w.T[0:HD]
    wo1 = attn_out_w.T[HD:2 * HD]
    bo = attn_out_b.reshape(1, HIDDEN_DIM)
    w1 = ff_w1.T
    b1 = ff_b1.reshape(1, -1)
    w2 = ff_w2.T
    b2 = ff_b2.reshape(1, -1)
    n1g = norm1_g.reshape(1, -1)
    n1b = norm1_b.reshape(1, -1)
    n2g = norm2_g.reshape(1, -1)
    n2b = norm2_b.reshape(1, -1)
    gate = gate_w.reshape(1, -1)
    wqp = qp_w.T
    bqp = qp_b.reshape(1, -1)
    wop = op_w.T
    bop = op_b.reshape(1, -1)

    full = lambda a: pl.BlockSpec(a.shape, lambda i: (0,) * a.ndim)
    vm_args = (embed, wq0, wq1, wk0, wk1, wv0, wv1, bq0, bq1, bk0, bk1,
               bv0, bv1, wo0, wo1, bo, w1, b1, w2, b2, n1g, n1b, n2g, n2b,
               gate, q_embed, wqp, bqp, wop, bop)
    out = pl.pallas_call(
        _step,
        grid=(B // BB,),
        in_specs=[pl.BlockSpec((BB * L, 1), lambda i: (i, 0)),
                  pl.BlockSpec((BB, 1), lambda i: (i, 0)),
                  pl.BlockSpec((BB, 1), lambda i: (i, 0)),
                  full(seg), full(seg2), full(eye_t)]
                 + [full(a) for a in vm_args],
        out_specs=pl.BlockSpec((1, 1), lambda i: (0, 0)),
        out_shape=jax.ShapeDtypeStruct((1, 1), f32),
        interpret=interpret,
    )(seq_col, query_col, target_col, seg, seg2, eye_t, *vm_args)
    return out[0, 0]


def kernel(seq, query, target, embed, in_proj_w, in_proj_b, attn_out_w,
           attn_out_b, ff_w1, ff_b1, ff_w2, ff_b2, norm1_g, norm1_b, norm2_g,
           norm2_b, gate_w, gate_b, q_embed, qp_w, qp_b, op_w, op_b):
    return _run(seq, query, target, embed, in_proj_w, in_proj_b, attn_out_w,
                attn_out_b, ff_w1, ff_b1, ff_w2, ff_b2, norm1_g, norm1_b,
                norm2_g, norm2_b, gate_w, gate_b, q_embed, qp_w, qp_b,
                op_w, op_b)


# speedup vs baseline: 1.2257x; 1.2257x over previous
import functools

import jax
import jax.numpy as jnp
from jax import lax
from jax.experimental import pallas as pl
from jax.experimental.pallas import tpu as pltpu
from jax.experimental.pallas import tpu_sc as plsc

HIDDEN_DIM = 64
MEMORY_SLOTS = 8
VOCAB_SIZE = 64
N_HEADS = 2
HEAD_DIM = HIDDEN_DIM // N_HEADS
B = 128
L = 512
BB = 32
V = VOCAB_SIZE

_TRANS_RHS = (((1,), (1,)), ((), ()))


def _dot(a, b):
    return jax.lax.dot_general(a, b, (((1,), (0,)), ((), ())),
                               preferred_element_type=jnp.float32)


def _dot_tb(a, b):
    return jax.lax.dot_general(a, b, _TRANS_RHS,
                               preferred_element_type=jnp.float32)


def _layer_norm(x, g, b):
    m = jnp.mean(x, axis=-1, keepdims=True)
    v = jnp.mean((x - m) ** 2, axis=-1, keepdims=True)
    return (x - m) * jax.lax.rsqrt(v + 1e-5) * g + b


def _sc_histogram(seq_flat):
    info = plsc.get_sparse_core_info()
    NC, NS = info.num_cores, info.num_subcores
    NW = NC * NS
    per_w = (B * L) // NW
    spw = B // NW
    vps = L // 16
    mesh = plsc.VectorSubcoreMesh(core_axis_name="c", subcore_axis_name="s")

    @functools.partial(
        pl.kernel, mesh=mesh,
        out_type=jax.ShapeDtypeStruct((B * V,), jnp.float32),
        scratch_types=[pltpu.VMEM((per_w,), jnp.int32),
                       pltpu.VMEM((spw * V,), jnp.float32)],
        compiler_params=pltpu.CompilerParams(needs_layout_passes=False),
    )
    def hist(seq_hbm, out_hbm, seq_v, cnt_v):
        wid = lax.axis_index("s") * NC + lax.axis_index("c")
        pltpu.sync_copy(seq_hbm.at[pl.ds(wid * per_w, per_w)], seq_v)
        zeros = jnp.zeros((16,), jnp.float32)
        for j in range(spw * V // 16):
            cnt_v[pl.ds(j * 16, 16)] = zeros
        ones = jnp.ones((16,), jnp.float32)
        for s in range(spw):
            for j in range(vps):
                vec = seq_v[pl.ds((s * vps + j) * 16, 16)]
                plsc.addupdate_scatter(cnt_v, [vec + s * V], ones)
        pltpu.sync_copy(cnt_v, out_hbm.at[pl.ds(wid * spw * V, spw * V)])

    return hist(seq_flat)


def _expand(mat):
    return jnp.reshape(
        jnp.broadcast_to(jnp.reshape(mat, (BB, 1, V)), (BB, V, V)),
        (BB * V, V))


def _step(c_ref, query_ref, target_ref, seg2_ref, eye_t_ref,
          embed_ref,
          wq0_ref, wq1_ref, wk0_ref, wk1_ref, wv0_ref, wv1_ref,
          bq0_ref, bq1_ref, bk0_ref, bk1_ref, bv0_ref, bv1_ref,
          wo0_ref, wo1_ref, bo_ref, w1_ref, b1_ref, w2_ref, b2_ref,
          n1g_ref, n1b_ref, n2g_ref, n2b_ref, gate_ref,
          qemb_ref, wqp_ref, bqp_ref, wop_ref, bop_ref, out_ref):
    i = pl.program_id(0)

    @pl.when(i == 0)
    def _():
        out_ref[...] = jnp.zeros_like(out_ref)

    BV = BB * V
    f32 = jnp.float32
    iota_bv = jax.lax.broadcasted_iota(jnp.int32, (BB, V), 1)
    ones_col = jnp.ones((V, 1), f32)
    eye_t = eye_t_ref[...]
    seg2 = seg2_ref[...]

    def rowsum(x):
        return _dot(x, ones_col)

    c_mat = c_ref[...]
    c_expand = _expand(c_mat)
    c_col = rowsum(c_expand * eye_t)

    emb = embed_ref[...]
    scale = 1.0 / (HEAD_DIM ** 0.5)

    def head_tab(wq, wk, bq, bk):
        q = _dot(emb, wq[...]) + bq[...]
        k = _dot(emb, wk[...]) + bk[...]
        return jnp.exp(_dot_tb(q * scale, k))

    expS0 = head_tab(wq0_ref, wk0_ref, bq0_ref, bk0_ref)
    expS1 = head_tab(wq1_ref, wk1_ref, bq1_ref, bk1_ref)
    v0 = _dot(emb, wv0_ref[...]) + bv0_ref[...]
    v1 = _dot(emb, wv1_ref[...]) + bv1_ref[...]

    def att(expS, vh):
        p = jnp.concatenate([expS] * BB, axis=0) * c_expand
        return _dot(p, vh) * (1.0 / rowsum(p))

    a_out = (_dot(att(expS0, v0), wo0_ref[...])
             + _dot(att(expS1, v1), wo1_ref[...]) + bo_ref[...])

    h0 = jnp.concatenate([emb] * BB, axis=0)
    h1 = _layer_norm(h0 + a_out, n1g_ref[...], n1b_ref[...])
    ff = _dot(jnp.maximum(_dot(h1, w1_ref[...]) + b1_ref[...], 0.0),
              w2_ref[...]) + b2_ref[...]
    h2 = _layer_norm(h1 + ff, n2g_ref[...], n2b_ref[...])

    s_col = rowsum(h2 * gate_ref[...])
    s_mat = _dot(seg2, s_col * eye_t)
    neg = jnp.float32(-3e38)
    sm_col = jnp.where(c_col > 0.0, s_col, neg)
    sm_exp = _expand(jnp.where(c_mat > 0.0, s_mat, neg))
    gt = (sm_exp > sm_col).astype(f32)
    before = rowsum(gt * c_expand)
    m_col = jnp.clip(jnp.float32(MEMORY_SLOTS) - before, 0.0, c_col)
    m_col = jnp.where(c_col > 0.0, m_col, 0.0)

    q_oh = (iota_bv == query_ref[...]).astype(f32)
    q_h = _dot(q_oh, qemb_ref[...])
    qp = _dot(q_h, wqp_ref[...]) + bqp_ref[...]
    s2_col = rowsum(h2 * _expand(qp)) * (1.0 / (HIDDEN_DIM ** 0.5))
    e2 = jnp.exp(s2_col) * m_col
    denom = _dot(seg2, e2)
    read = _dot(seg2, e2 * h2) * (1.0 / denom)

    logits = _dot(read, wop_ref[...]) + bop_ref[...]
    ml = jnp.max(logits, axis=1, keepdims=True)
    lse = ml + jnp.log(jnp.sum(jnp.exp(logits - ml), axis=1, keepdims=True))
    t_oh = (iota_bv == target_ref[...]).astype(f32)
    tgt = jnp.sum(logits * t_oh, axis=1, keepdims=True)
    out_ref[...] += jnp.sum(lse - tgt) * (1.0 / B)


@functools.partial(jax.jit, static_argnames=("interpret",))
def _run(seq, query, target, embed, in_proj_w, in_proj_b, attn_out_w,
         attn_out_b, ff_w1, ff_b1, ff_w2, ff_b2, norm1_g, norm1_b, norm2_g,
         norm2_b, gate_w, gate_b, q_embed, qp_w, qp_b, op_w, op_b,
         interpret=False):
    f32 = jnp.float32
    c_all = _sc_histogram(seq.astype(jnp.int32).reshape(B * L)).reshape(B, V)
    query_col = query.astype(jnp.int32).reshape(B, 1)
    target_col = target.astype(jnp.int32).reshape(B, 1)
    BV = BB * V
    seg2 = (jnp.arange(BV, dtype=jnp.int32)[None, :] // V
            == jnp.arange(BB, dtype=jnp.int32)[:, None]).astype(f32)
    eye_t = jnp.tile(jnp.eye(V, dtype=f32), (BB, 1))
    HD = HEAD_DIM
    wq0 = in_proj_w[0:HD].T
    wq1 = in_proj_w[HD:2 * HD].T
    wk0 = in_proj_w[2 * HD:3 * HD].T
    wk1 = in_proj_w[3 * HD:4 * HD].T
    wv0 = in_proj_w[4 * HD:5 * HD].T
    wv1 = in_proj_w[5 * HD:6 * HD].T
    bq0 = in_proj_b[0:HD].reshape(1, HD)
    bq1 = in_proj_b[HD:2 * HD].reshape(1, HD)
    bk0 = in_proj_b[2 * HD:3 * HD].reshape(1, HD)
    bk1 = in_proj_b[3 * HD:4 * HD].reshape(1, HD)
    bv0 = in_proj_b[4 * HD:5 * HD].reshape(1, HD)
    bv1 = in_proj_b[5 * HD:6 * HD].reshape(1, HD)
    wo0 = attn_out_w.T[0:HD]
    wo1 = attn_out_w.T[HD:2 * HD]
    bo = attn_out_b.reshape(1, HIDDEN_DIM)
    w1 = ff_w1.T
    b1 = ff_b1.reshape(1, -1)
    w2 = ff_w2.T
    b2 = ff_b2.reshape(1, -1)
    n1g = norm1_g.reshape(1, -1)
    n1b = norm1_b.reshape(1, -1)
    n2g = norm2_g.reshape(1, -1)
    n2b = norm2_b.reshape(1, -1)
    gate = gate_w.reshape(1, -1)
    wqp = qp_w.T
    bqp = qp_b.reshape(1, -1)
    wop = op_w.T
    bop = op_b.reshape(1, -1)

    full = lambda a: pl.BlockSpec(a.shape, lambda i: (0,) * a.ndim)
    vm_args = (embed, wq0, wq1, wk0, wk1, wv0, wv1, bq0, bq1, bk0, bk1,
               bv0, bv1, wo0, wo1, bo, w1, b1, w2, b2, n1g, n1b, n2g, n2b,
               gate, q_embed, wqp, bqp, wop, bop)
    out = pl.pallas_call(
        _step,
        grid=(B // BB,),
        in_specs=[pl.BlockSpec((BB, V), lambda i: (i, 0)),
                  pl.BlockSpec((BB, 1), lambda i: (i, 0)),
                  pl.BlockSpec((BB, 1), lambda i: (i, 0)),
                  full(seg2), full(eye_t)]
                 + [full(a) for a in vm_args],
        out_specs=pl.BlockSpec((1, 1), lambda i: (0, 0)),
        out_shape=jax.ShapeDtypeStruct((1, 1), f32),
        interpret=interpret,
    )(c_all, query_col, target_col, seg2, eye_t, *vm_args)
    return out[0, 0]


def kernel(seq, query, target, embed, in_proj_w, in_proj_b, attn_out_w,
           attn_out_b, ff_w1, ff_b1, ff_w2, ff_b2, norm1_g, norm1_b, norm2_g,
           norm2_b, gate_w, gate_b, q_embed, qp_w, qp_b, op_w, op_b):
    return _run(seq, query, target, embed, in_proj_w, in_proj_b, attn_out_w,
                attn_out_b, ff_w1, ff_b1, ff_w2, ff_b2, norm1_g, norm1_b,
                norm2_g, norm2_b, gate_w, gate_b, q_embed, qp_w, qp_b,
                op_w, op_b)
